# initial kernel scaffold (unmeasured)
import jax
import jax.numpy as jnp
from jax import lax
from jax.experimental import pallas as pl
from jax.experimental.pallas import tpu as pltpu


def kernel(x, W):
    m, k = x.shape
    _, v_loc = W.shape

    NB = 1024

    def gemm_body(x_ref, w_ref, out_ref):
        out_ref[...] = jnp.dot(
            x_ref[...].astype(jnp.bfloat16),
            w_ref[...].astype(jnp.bfloat16),
            preferred_element_type=jnp.float32,
        ).astype(jnp.bfloat16)

    logits = pl.pallas_call(
        gemm_body,
        grid=(v_loc // NB,),
        in_specs=[
            pl.BlockSpec((m, k), lambda j: (0, 0)),
            pl.BlockSpec((k, NB), lambda j: (0, j)),
        ],
        out_specs=pl.BlockSpec((m, NB), lambda j: (0, j)),
        out_shape=jax.ShapeDtypeStruct((m, v_loc), jnp.bfloat16),
    )(x, W)

    def exch_body(l_ref, full_ref, copy_sem, send_sem, recv_sem):
        my_x = lax.axis_index("x")
        my_y = lax.axis_index("y")
        local_cp = pltpu.make_async_copy(l_ref, full_ref.at[my_x], copy_sem)
        local_cp.start()
        rdma = pltpu.make_async_remote_copy(
            src_ref=l_ref,
            dst_ref=full_ref.at[my_x],
            send_sem=send_sem,
            recv_sem=recv_sem,
            device_id=(1 - my_x, my_y),
            device_id_type=pl.DeviceIdType.MESH,
        )
        rdma.start()
        local_cp.wait()
        rdma.wait()

    full = pl.pallas_call(
        exch_body,
        in_specs=[pl.BlockSpec(memory_space=pltpu.ANY)],
        out_specs=pl.BlockSpec(memory_space=pltpu.ANY),
        out_shape=jax.ShapeDtypeStruct((2, m, v_loc), jnp.bfloat16),
        scratch_shapes=[
            pltpu.SemaphoreType.DMA,
            pltpu.SemaphoreType.DMA,
            pltpu.SemaphoreType.DMA,
        ],
        compiler_params=pltpu.CompilerParams(has_side_effects=True),
    )(logits)

    RB = 64

    def sm_body(full_ref, out_ref):
        l0 = full_ref[0].astype(jnp.float32)
        l1 = full_ref[1].astype(jnp.float32)
        mx = jnp.maximum(
            l0.max(axis=1, keepdims=True), l1.max(axis=1, keepdims=True)
        )
        e0 = jnp.exp(l0 - mx)
        e1 = jnp.exp(l1 - mx)
        s = e0.sum(axis=1, keepdims=True) + e1.sum(axis=1, keepdims=True)
        out_ref[:, :v_loc] = e0 / s
        out_ref[:, v_loc:] = e1 / s

    return pl.pallas_call(
        sm_body,
        grid=(m // RB,),
        in_specs=[pl.BlockSpec((2, RB, v_loc), lambda i: (0, i, 0))],
        out_specs=pl.BlockSpec((RB, 2 * v_loc), lambda i: (i, 0)),
        out_shape=jax.ShapeDtypeStruct((m, 2 * v_loc), jnp.float32),
    )(full)


# baseline (device time: 1177659 ns/iter reference)
import jax
import jax.numpy as jnp
from jax import lax
from jax.experimental import pallas as pl
from jax.experimental.pallas import tpu as pltpu


def kernel(x, W):
    m, k = x.shape
    _, v_loc = W.shape

    NB = 1024

    def gemm_body(x_ref, w_ref, out_ref):
        out_ref[...] = jnp.dot(
            x_ref[...].astype(jnp.bfloat16),
            w_ref[...].astype(jnp.bfloat16),
            preferred_element_type=jnp.float32,
        ).astype(jnp.bfloat16)

    logits = pl.pallas_call(
        gemm_body,
        grid=(v_loc // NB,),
        in_specs=[
            pl.BlockSpec((m, k), lambda j: (0, 0)),
            pl.BlockSpec((k, NB), lambda j: (0, j)),
        ],
        out_specs=pl.BlockSpec((m, NB), lambda j: (0, j)),
        out_shape=jax.ShapeDtypeStruct((m, v_loc), jnp.bfloat16),
    )(x, W)

    def exch_body(l_ref, full_ref, copy_sem, send_sem, recv_sem):
        my_x = lax.axis_index("x")
        my_y = lax.axis_index("y")
        local_cp = pltpu.make_async_copy(l_ref, full_ref.at[my_x], copy_sem)
        local_cp.start()
        rdma = pltpu.make_async_remote_copy(
            src_ref=l_ref,
            dst_ref=full_ref.at[my_x],
            send_sem=send_sem,
            recv_sem=recv_sem,
            device_id=(1 - my_x, my_y),
            device_id_type=pl.DeviceIdType.MESH,
        )
        rdma.start()
        local_cp.wait()
        rdma.wait()

    full = pl.pallas_call(
        exch_body,
        in_specs=[pl.BlockSpec(memory_space=pl.ANY)],
        out_specs=pl.BlockSpec(memory_space=pl.ANY),
        out_shape=jax.ShapeDtypeStruct((2, m, v_loc), jnp.bfloat16),
        scratch_shapes=[
            pltpu.SemaphoreType.DMA,
            pltpu.SemaphoreType.DMA,
            pltpu.SemaphoreType.DMA,
        ],
        compiler_params=pltpu.CompilerParams(has_side_effects=True),
    )(logits)

    RB = 64

    def sm_body(full_ref, out_ref):
        l0 = full_ref[0].astype(jnp.float32)
        l1 = full_ref[1].astype(jnp.float32)
        mx = jnp.maximum(
            l0.max(axis=1, keepdims=True), l1.max(axis=1, keepdims=True)
        )
        e0 = jnp.exp(l0 - mx)
        e1 = jnp.exp(l1 - mx)
        s = e0.sum(axis=1, keepdims=True) + e1.sum(axis=1, keepdims=True)
        out_ref[:, :v_loc] = e0 / s
        out_ref[:, v_loc:] = e1 / s

    return pl.pallas_call(
        sm_body,
        grid=(m // RB,),
        in_specs=[pl.BlockSpec((2, RB, v_loc), lambda i: (0, i, 0))],
        out_specs=pl.BlockSpec((RB, 2 * v_loc), lambda i: (i, 0)),
        out_shape=jax.ShapeDtypeStruct((m, 2 * v_loc), jnp.float32),
    )(full)


# device time: 447789 ns/iter; 2.6299x vs baseline; 2.6299x over previous
import jax
import jax.numpy as jnp
from jax import lax
from jax.experimental import pallas as pl
from jax.experimental.pallas import tpu as pltpu


def kernel(x, W):
    m, k = x.shape
    _, v_loc = W.shape

    NB = 1024
    NCHUNK = v_loc // NB
    NSLOT = 4

    def fused_body(x_ref, w_ref, full_ref, lblk, copy_sems, send_sems, recv_sems):
        j = pl.program_id(0)
        my_x = lax.axis_index("x")
        my_y = lax.axis_index("y")

        def chunk_copy(jj, sem):
            return pltpu.make_async_copy(
                lblk.at[lax.rem(jj, NSLOT)],
                full_ref.at[my_x, :, pl.ds(jj * NB, NB)],
                sem,
            )

        def chunk_rdma(jj, send_sem, recv_sem):
            return pltpu.make_async_remote_copy(
                src_ref=lblk.at[lax.rem(jj, NSLOT)],
                dst_ref=full_ref.at[my_x, :, pl.ds(jj * NB, NB)],
                send_sem=send_sem,
                recv_sem=recv_sem,
                device_id=(1 - my_x, my_y),
                device_id_type=pl.DeviceIdType.MESH,
            )

        @pl.when(j >= NSLOT)
        def _():
            chunk_copy(j - NSLOT, copy_sems.at[j - NSLOT]).wait()
            chunk_rdma(
                j - NSLOT, send_sems.at[j - NSLOT], recv_sems.at[j - NSLOT]
            ).wait_send()

        lblk[lax.rem(j, NSLOT)] = jnp.dot(
            x_ref[...].astype(jnp.bfloat16),
            w_ref[...].astype(jnp.bfloat16),
            preferred_element_type=jnp.float32,
        ).astype(jnp.bfloat16)

        chunk_copy(j, copy_sems.at[j]).start()
        chunk_rdma(j, send_sems.at[j], recv_sems.at[j]).start()

        @pl.when(j == NCHUNK - 1)
        def _():
            for jj in range(NCHUNK - NSLOT, NCHUNK):
                chunk_copy(jj, copy_sems.at[jj]).wait()
                chunk_rdma(jj, send_sems.at[jj], recv_sems.at[jj]).wait_send()
            for jj in range(NCHUNK):
                chunk_rdma(jj, send_sems.at[jj], recv_sems.at[jj]).wait_recv()

    full = pl.pallas_call(
        fused_body,
        grid=(NCHUNK,),
        in_specs=[
            pl.BlockSpec((m, k), lambda j: (0, 0)),
            pl.BlockSpec((k, NB), lambda j: (0, j)),
        ],
        out_specs=pl.BlockSpec(memory_space=pl.ANY),
        out_shape=jax.ShapeDtypeStruct((2, m, v_loc), jnp.bfloat16),
        scratch_shapes=[
            pltpu.VMEM((NSLOT, m, NB), jnp.bfloat16),
            pltpu.SemaphoreType.DMA((NCHUNK,)),
            pltpu.SemaphoreType.DMA((NCHUNK,)),
            pltpu.SemaphoreType.DMA((NCHUNK,)),
        ],
        compiler_params=pltpu.CompilerParams(has_side_effects=True),
    )(x, W)

    RB = 64

    def sm_body(full_ref, out_ref):
        l0 = full_ref[0].astype(jnp.float32)
        l1 = full_ref[1].astype(jnp.float32)
        mx = jnp.maximum(
            l0.max(axis=1, keepdims=True), l1.max(axis=1, keepdims=True)
        )
        e0 = jnp.exp(l0 - mx)
        e1 = jnp.exp(l1 - mx)
        s = e0.sum(axis=1, keepdims=True) + e1.sum(axis=1, keepdims=True)
        out_ref[:, :v_loc] = e0 / s
        out_ref[:, v_loc:] = e1 / s

    return pl.pallas_call(
        sm_body,
        grid=(m // RB,),
        in_specs=[pl.BlockSpec((2, RB, v_loc), lambda i: (0, i, 0))],
        out_specs=pl.BlockSpec((RB, 2 * v_loc), lambda i: (i, 0)),
        out_shape=jax.ShapeDtypeStruct((m, 2 * v_loc), jnp.float32),
    )(full)
